# baseline (speedup 1.0000x reference)
import functools

import jax
import jax.numpy as jnp
from jax import lax
from jax.experimental import pallas as pl
from jax.experimental.pallas import tpu as pltpu
from jax.experimental.pallas import tpu_sc as plsc

H = 26
D = 64
DP = 128
BLK = 128
L = 16
NP = 13
RP = 200000


@functools.lru_cache(maxsize=None)
def _build(b, pc):
    info = plsc.get_sparse_core_info()
    nc, ns = info.num_cores, info.num_subcores
    nw = nc * ns
    nblk = b // BLK
    units = 2 * nblk
    per_w = units // nw
    assert per_w * nw == units and per_w % 2 == 0

    mesh = plsc.VectorSubcoreMesh(core_axis_name="c", subcore_axis_name="s")

    @functools.partial(
        pl.kernel,
        mesh=mesh,
        out_type=jax.ShapeDtypeStruct((2, D, b), jnp.float32),
        compiler_params=pltpu.CompilerParams(
            use_tc_tiling_on_sc=True, needs_layout_passes=False),
        scratch_types=[
            pltpu.VMEM((32,), jnp.int32),
            pltpu.VMEM((BLK,), jnp.int32),
            pltpu.VMEM((2, BLK), jnp.int32),
            pltpu.VMEM((2, BLK, DP), jnp.float32),
            pltpu.VMEM((2, D, BLK), jnp.float32),
            pltpu.SemaphoreType.DMA,
            pltpu.SemaphoreType.DMA,
            pltpu.SemaphoreType.DMA,
            pltpu.SemaphoreType.DMA,
        ],
    )
    def k(ids_hbm, table_hbm, off_hbm, out_hbm,
          off_v, ids_v, row_v, rows_v, out_t,
          g0, g1, w0, w1):
        wid = lax.axis_index("s") * nc + lax.axis_index("c")
        u0 = wid * per_w
        pltpu.sync_copy(off_hbm, off_v.at[pl.ds(0, H)])
        iota = lax.broadcasted_iota(jnp.int32, (L,), 0)
        rots = [(iota + k2) & (L - 1) for k2 in range(L)]
        gsem = (g0, g1)
        wsem = (w0, w1)

        def coords(u):
            return u // nblk, (u % nblk) * BLK

        def make_idx(u, p):
            hl, b0 = coords(u)
            h = 2 * pc + hl
            pltpu.sync_copy(ids_hbm.at[h, pl.ds(b0, BLK)], ids_v)
            off16 = plsc.load_gather(off_v, [jnp.broadcast_to(h, (L,))])
            adj = off16 - (RP * pc)
            for r in range(BLK // L):
                sl = pl.ds(r * L, L)
                row_v[p, sl] = ids_v[sl] + adj

        def start_gather(p):
            return pltpu.async_copy(
                table_hbm.at[row_v.at[p]], rows_v.at[p], gsem[p])

        def wait_gather(p):
            pltpu.make_async_copy(
                table_hbm.at[row_v.at[p]], rows_v.at[p], gsem[p]).wait()

        def start_write(u, p):
            hl, b0 = coords(u)
            return pltpu.async_copy(
                out_t.at[p], out_hbm.at[hl, :, pl.ds(b0, BLK)], wsem[p])

        def wait_write(u, p):
            hl, b0 = coords(u)
            pltpu.make_async_copy(
                out_t.at[p], out_hbm.at[hl, :, pl.ds(b0, BLK)], wsem[p]).wait()

        def compact(p):
            def block(g, carry):
                i0 = g * L
                for dc in range(D // L):
                    d0 = dc * L
                    for kk in range(L):
                        v = plsc.load_gather(
                            rows_v.at[p], [i0 + rots[kk], d0 + iota])
                        plsc.store_scatter(
                            out_t.at[p], [d0 + iota, i0 + rots[kk]], v)
                return carry
            lax.fori_loop(0, BLK // L, block, 0)

        make_idx(u0, 0)
        start_gather(0)

        def pair_body(j, carry):
            for p in (0, 1):
                u = u0 + 2 * j + p
                wait_gather(p)
                if p == 0:
                    make_idx(u + 1, 1)
                    start_gather(1)
                else:
                    @pl.when(j < per_w // 2 - 1)
                    def _():
                        make_idx(u + 1, 0)
                        start_gather(0)
                @pl.when(j > 0)
                def _():
                    wait_write(u - 2, p)
                compact(p)
                start_write(u, p)
            return carry

        lax.fori_loop(0, per_w // 2, pair_body, 0)
        wait_write(u0 + per_w - 2, 0)
        wait_write(u0 + per_w - 1, 1)

    return k


def kernel(input_ids, table, offsets):
    b, h = input_ids.shape
    ids_t = input_ids.T
    outs = []
    for pc in range(NP):
        piece = jnp.pad(table[pc * RP:(pc + 1) * RP], ((0, 0), (0, DP - D)))
        outs.append(_build(b, pc)(ids_t, piece, offsets))
    outk = jnp.concatenate(outs, axis=0)
    return jnp.transpose(outk, (2, 0, 1))


# speedup vs baseline: 1.3905x; 1.3905x over previous
import functools

import jax
import jax.numpy as jnp
from jax import lax
from jax.experimental import pallas as pl
from jax.experimental.pallas import tpu as pltpu
from jax.experimental.pallas import tpu_sc as plsc

H = 26
D = 64
DP = 128
BLK = 128
L = 16


@functools.lru_cache(maxsize=None)
def _build(b):
    info = plsc.get_sparse_core_info()
    nc, ns = info.num_cores, info.num_subcores
    nw = nc * ns
    nblk = b // BLK
    units = H * nblk
    per_w = units // nw
    assert per_w * nw == units and per_w % 2 == 0

    mesh = plsc.VectorSubcoreMesh(core_axis_name="c", subcore_axis_name="s")

    @functools.partial(
        pl.kernel,
        mesh=mesh,
        out_type=jax.ShapeDtypeStruct((H, D, b), jnp.float32),
        compiler_params=pltpu.CompilerParams(
            use_tc_tiling_on_sc=True, needs_layout_passes=False),
        scratch_types=[
            pltpu.VMEM((32,), jnp.int32),
            pltpu.VMEM((BLK,), jnp.int32),
            pltpu.VMEM((2, BLK), jnp.int32),
            pltpu.VMEM((2, BLK, DP), jnp.float32),
            pltpu.VMEM((2, D, BLK), jnp.float32),
            pltpu.SemaphoreType.DMA,
            pltpu.SemaphoreType.DMA,
            pltpu.SemaphoreType.DMA,
            pltpu.SemaphoreType.DMA,
        ],
    )
    def k(ids_hbm, table_hbm, off_hbm, out_hbm,
          off_v, ids_v, row_v, rows_v, out_t,
          g0, g1, w0, w1):
        wid = lax.axis_index("s") * nc + lax.axis_index("c")
        u0 = wid * per_w
        pltpu.sync_copy(off_hbm, off_v.at[pl.ds(0, H)])
        iota = lax.broadcasted_iota(jnp.int32, (L,), 0)
        rots = [(iota + k) & (L - 1) for k in range(L)]
        gsem = (g0, g1)
        wsem = (w0, w1)

        def coords(u):
            return u // nblk, (u % nblk) * BLK

        def make_idx(u, p):
            h, b0 = coords(u)
            pltpu.sync_copy(ids_hbm.at[h, pl.ds(b0, BLK)], ids_v)
            off16 = plsc.load_gather(off_v, [jnp.broadcast_to(h, (L,))])
            for r in range(BLK // L):
                sl = pl.ds(r * L, L)
                row_v[p, sl] = ids_v[sl] + off16

        def start_gather(p):
            return pltpu.async_copy(
                table_hbm.at[row_v.at[p]], rows_v.at[p], gsem[p])

        def wait_gather(p):
            pltpu.make_async_copy(
                table_hbm.at[row_v.at[p]], rows_v.at[p], gsem[p]).wait()

        def start_write(u, p):
            h, b0 = coords(u)
            return pltpu.async_copy(
                out_t.at[p], out_hbm.at[h, :, pl.ds(b0, BLK)], wsem[p])

        def wait_write(u, p):
            h, b0 = coords(u)
            pltpu.make_async_copy(
                out_t.at[p], out_hbm.at[h, :, pl.ds(b0, BLK)], wsem[p]).wait()

        def compact(p):
            def block(g, carry):
                i0 = g * L
                for dc in range(D // L):
                    d0 = dc * L
                    for kk in range(L):
                        v = plsc.load_gather(
                            rows_v.at[p], [i0 + rots[kk], d0 + iota])
                        plsc.store_scatter(
                            out_t.at[p], [d0 + iota, i0 + rots[kk]], v)
                return carry
            lax.fori_loop(0, BLK // L, block, 0)

        make_idx(u0, 0)
        start_gather(0)

        def pair_body(j, carry):
            for p in (0, 1):
                u = u0 + 2 * j + p
                wait_gather(p)
                if p == 0:
                    make_idx(u + 1, 1)
                    start_gather(1)
                else:
                    @pl.when(j < per_w // 2 - 1)
                    def _():
                        make_idx(u + 1, 0)
                        start_gather(0)
                @pl.when(j > 0)
                def _():
                    wait_write(u - 2, p)
                compact(p)
                start_write(u, p)
            return carry

        lax.fori_loop(0, per_w // 2, pair_body, 0)
        wait_write(u0 + per_w - 2, 0)
        wait_write(u0 + per_w - 1, 1)

    return k


def kernel(input_ids, table, offsets):
    b, h = input_ids.shape
    ids_t = input_ids.T
    table_p = jnp.pad(table, ((0, 0), (0, DP - D)))
    outk = _build(b)(ids_t, table_p, offsets)
    return jnp.transpose(outk, (2, 0, 1))
